# baseline (device time: 50657 ns/iter reference)
import jax
import jax.numpy as jnp
from jax import lax
from jax.experimental import pallas as pl
from jax.experimental.pallas import tpu as pltpu

C = 8
LAG = 2
DN = (((1,), (1,)), ((), ()))


def kernel(dy, W):
    m, k = dy.shape
    n, k2 = W.shape
    assert k == k2
    h = n // 2
    r = m // C

    def body(dy_hbm, w_hbm, out_ref, wxbuf, dybuf, pbuf, ybuf, red, xbuf,
             wx_sem, dy_sems, ysend, yrecv, xsend, xrecv, out_sems):
        my_x = lax.axis_index("x")
        my_y = lax.axis_index("y")
        my_z = lax.axis_index("z")
        ypartner = (my_x, 1 - my_y, my_z)
        xpartner = (1 - my_x, my_y, my_z)

        wx_dma = pltpu.make_async_copy(
            w_hbm.at[pl.ds(my_x * h, h), :], wxbuf, wx_sem)
        wx_dma.start()
        dy_dmas = []
        for c in range(C):
            d = pltpu.make_async_copy(
                dy_hbm.at[pl.ds(c * r, r), :], dybuf.at[c], dy_sems.at[c])
            d.start()
            dy_dmas.append(d)

        barrier_sem = pltpu.get_barrier_semaphore()
        for nbr in (ypartner, xpartner):
            pl.semaphore_signal(
                barrier_sem, inc=1,
                device_id=nbr, device_id_type=pl.DeviceIdType.MESH,
            )
        pl.semaphore_wait(barrier_sem, 2)

        wx_dma.wait()
        y_rdmas = []
        for c in range(C):
            dy_dmas[c].wait()
            pbuf[c] = lax.dot_general(
                dybuf[c], wxbuf[...], DN,
                preferred_element_type=jnp.float32,
            )
            rdma = pltpu.make_async_remote_copy(
                src_ref=pbuf.at[c], dst_ref=ybuf.at[c],
                send_sem=ysend.at[c], recv_sem=yrecv.at[c],
                device_id=ypartner, device_id_type=pl.DeviceIdType.MESH,
            )
            rdma.start()
            y_rdmas.append(rdma)

        x_rdmas = []
        out_dmas = []

        def drain_x(c):
            x_rdmas[c].wait_recv()
            d = pltpu.make_async_copy(
                xbuf.at[c],
                out_ref.at[pl.ds(c * r, r), pl.ds((1 - my_x) * h, h)],
                out_sems.at[C + c])
            d.start()
            out_dmas.append(d)

        for c in range(C):
            y_rdmas[c].wait_recv()
            red[c] = pbuf[c] + ybuf[c]
            rdma = pltpu.make_async_remote_copy(
                src_ref=red.at[c], dst_ref=xbuf.at[c],
                send_sem=xsend.at[c], recv_sem=xrecv.at[c],
                device_id=xpartner, device_id_type=pl.DeviceIdType.MESH,
            )
            rdma.start()
            x_rdmas.append(rdma)
            d = pltpu.make_async_copy(
                red.at[c],
                out_ref.at[pl.ds(c * r, r), pl.ds(my_x * h, h)],
                out_sems.at[c])
            d.start()
            out_dmas.append(d)
            if c >= LAG:
                drain_x(c - LAG)

        for c in range(C - LAG, C):
            drain_x(c)
        for c in range(C):
            y_rdmas[c].wait_send()
            x_rdmas[c].wait_send()
        for d in out_dmas:
            d.wait()

    return pl.pallas_call(
        body,
        out_shape=jax.ShapeDtypeStruct((m, n), jnp.float32),
        in_specs=[
            pl.BlockSpec(memory_space=pltpu.MemorySpace.HBM),
            pl.BlockSpec(memory_space=pltpu.MemorySpace.HBM),
        ],
        out_specs=pl.BlockSpec(memory_space=pltpu.MemorySpace.HBM),
        scratch_shapes=[
            pltpu.VMEM((h, k), jnp.float32),
            pltpu.VMEM((C, r, k), jnp.float32),
            pltpu.VMEM((C, r, h), jnp.float32),
            pltpu.VMEM((C, r, h), jnp.float32),
            pltpu.VMEM((C, r, h), jnp.float32),
            pltpu.VMEM((C, r, h), jnp.float32),
            pltpu.SemaphoreType.DMA,
            pltpu.SemaphoreType.DMA((C,)),
            pltpu.SemaphoreType.DMA((C,)),
            pltpu.SemaphoreType.DMA((C,)),
            pltpu.SemaphoreType.DMA((C,)),
            pltpu.SemaphoreType.DMA((C,)),
            pltpu.SemaphoreType.DMA((2 * C,)),
        ],
        compiler_params=pltpu.CompilerParams(collective_id=0),
    )(dy, W)
